# Initial kernel scaffold; baseline (speedup 1.0000x reference)
import jax
import jax.numpy as jnp
from jax.experimental import pallas as pl


def kernel(x, edge_index, W1, b1, W2, b2, Wc1, bc1, Wc2, bc2):
    raise NotImplementedError("write your pallas kernel here")


# speedup vs baseline: 5.1166x; 5.1166x over previous
import functools

import jax
import jax.numpy as jnp
from jax import lax
from jax.experimental import pallas as pl
from jax.experimental.pallas import tpu as pltpu
from jax.experimental.pallas import tpu_sc as plsc

N = 10000
E = 320000
NC = 2
NS = 16
NW = NC * NS
EPW = E // NW
K = 80
NCHUNK = EPW // K
RPT = 632


def _make_edge_scatter(D):
  mesh = plsc.VectorSubcoreMesh(
      core_axis_name="c", subcore_axis_name="s", num_cores=NC, num_subcores=NS
  )

  @functools.partial(
      pl.kernel,
      out_type=jax.ShapeDtypeStruct((NC, N, D), jnp.float32),
      mesh=mesh,
      scratch_types=[
          pltpu.VMEM((NCHUNK, K), jnp.int32),
          pltpu.VMEM((NCHUNK, K), jnp.int32),
          pltpu.VMEM((K, D), jnp.float32),
          pltpu.VMEM_SHARED((N, D), jnp.float32),
          pltpu.SemaphoreType.DMA,
      ],
      compiler_params=pltpu.CompilerParams(use_tc_tiling_on_sc=False),
  )
  def k(table_hbm, src_hbm, dst_hbm, zeros_hbm, out_hbm,
        src_v, dst_v, rows_v, acc, sem):
    c = lax.axis_index("c")
    s = lax.axis_index("s")
    wid = s * NC + c
    pltpu.sync_copy(src_hbm.at[wid], src_v)
    pltpu.sync_copy(dst_hbm.at[wid], dst_v)
    row0 = jnp.minimum(s * RPT, N - RPT)
    pltpu.sync_copy(zeros_hbm, acc.at[pl.ds(row0, RPT)])
    plsc.subcore_barrier()

    def body(j, carry):
      pltpu.async_copy(table_hbm.at[src_v.at[j]], rows_v, sem).wait()
      pltpu.sync_copy(rows_v, acc.at[dst_v.at[j]], add=True)
      return carry

    lax.fori_loop(0, NCHUNK, body, 0)
    plsc.subcore_barrier()
    pltpu.sync_copy(acc.at[pl.ds(row0, RPT)],
                    out_hbm.at[c, pl.ds(row0, RPT)])

  return k


_scatter_deg = _make_edge_scatter(8)
_scatter_l1 = _make_edge_scatter(64)
_scatter_l2 = _make_edge_scatter(32)


_ROWS = 2000
_GRID = N // _ROWS


def _rb(d):
  return pl.BlockSpec((_ROWS, d), lambda i: (i, 0))


def _full(a, b):
  return pl.BlockSpec((a, b), lambda i: (0, 0))


def _tc_stage1_body(deg0_ref, deg1_ref, x_ref, w1_ref, dinv_ref, h1p_ref):
  deg = deg0_ref[:, :1] + deg1_ref[:, :1] + 1.0
  dinv = lax.rsqrt(deg)
  dinv_ref[...] = dinv
  h = lax.dot_general(x_ref[...], w1_ref[...], (((1,), (1,)), ((), ())),
                      preferred_element_type=jnp.float32)
  h1p_ref[...] = dinv * h


def _tc_stage1(deg0, deg1, x, W1):
  return pl.pallas_call(
      _tc_stage1_body,
      grid=(_GRID,),
      in_specs=[_rb(8), _rb(8), _rb(128), _full(64, 128)],
      out_specs=[_rb(1), _rb(64)],
      out_shape=[
          jax.ShapeDtypeStruct((N, 1), jnp.float32),
          jax.ShapeDtypeStruct((N, 64), jnp.float32),
      ],
  )(deg0, deg1, x, W1)


def _tc_stage2_body(p0_ref, p1_ref, h1p_ref, dinv_ref, b1_ref, w2_ref,
                    h2p_ref):
  dinv = dinv_ref[...]
  g = p0_ref[...] + p1_ref[...] + h1p_ref[...]
  h1 = jnp.maximum(dinv * g + b1_ref[...], 0.0)
  h2 = lax.dot_general(h1, w2_ref[...], (((1,), (1,)), ((), ())),
                       preferred_element_type=jnp.float32)
  h2p_ref[...] = dinv * h2


def _tc_stage2(p0, p1, h1p, dinv, b1, W2):
  return pl.pallas_call(
      _tc_stage2_body,
      grid=(_GRID,),
      in_specs=[_rb(64), _rb(64), _rb(64), _rb(1), _full(1, 64),
                _full(32, 64)],
      out_specs=_rb(32),
      out_shape=jax.ShapeDtypeStruct((N, 32), jnp.float32),
  )(p0, p1, h1p, dinv, b1, W2)


def _tc_stage3_body(p0_ref, p1_ref, h2p_ref, dinv_ref, b2_ref, wc1_ref,
                    bc1_ref, wc2_ref, bc2_ref, out_ref):
  g = p0_ref[...] + p1_ref[...] + h2p_ref[...]
  h2 = jnp.maximum(dinv_ref[...] * g + b2_ref[...], 0.0)
  hc = lax.dot_general(h2, wc1_ref[...], (((1,), (1,)), ((), ())),
                       preferred_element_type=jnp.float32)
  hc = jnp.maximum(hc + bc1_ref[...], 0.0)
  out = lax.dot_general(hc, wc2_ref[...], (((1,), (1,)), ((), ())),
                        preferred_element_type=jnp.float32)
  out_ref[...] = out + bc2_ref[...]


def _tc_stage3(p0, p1, h2p, dinv, b2, Wc1, bc1, Wc2, bc2):
  return pl.pallas_call(
      _tc_stage3_body,
      grid=(_GRID,),
      in_specs=[_rb(32), _rb(32), _rb(32), _rb(1), _full(1, 32),
                _full(16, 32), _full(1, 16), _full(10, 16), _full(1, 10)],
      out_specs=_rb(10),
      out_shape=jax.ShapeDtypeStruct((N, 10), jnp.float32),
  )(p0, p1, h2p, dinv, b2, Wc1, bc1, Wc2, bc2)


def kernel(x, edge_index, W1, b1, W2, b2, Wc1, bc1, Wc2, bc2):
  src3 = edge_index[0].reshape(NW, NCHUNK, K)
  dst3 = edge_index[1].reshape(NW, NCHUNK, K)
  zero_idx = jnp.zeros((NW, NCHUNK, K), jnp.int32)
  ones_tbl = jnp.ones((8, 8), jnp.float32)
  z8 = jnp.zeros((RPT, 8), jnp.float32)
  z64 = jnp.zeros((RPT, 64), jnp.float32)
  z32 = jnp.zeros((RPT, 32), jnp.float32)

  degp = _scatter_deg(ones_tbl, zero_idx, dst3, z8)
  dinv, h1p = _tc_stage1(degp[0], degp[1], x, W1)
  p1 = _scatter_l1(h1p, src3, dst3, z64)
  h2p = _tc_stage2(p1[0], p1[1], h1p, dinv, b1.reshape(1, 64), W2)
  p2 = _scatter_l2(h2p, src3, dst3, z32)
  out = _tc_stage3(p2[0], p2[1], h2p, dinv, b2.reshape(1, 32),
                   Wc1, bc1.reshape(1, 16), Wc2, bc2.reshape(1, 10))
  return out
